# baseline (speedup 1.0000x reference)
import functools

import jax
import jax.numpy as jnp
from jax import lax
from jax.experimental import pallas as pl
from jax.experimental.pallas import tpu as pltpu
from jax.experimental.pallas import tpu_sc as plsc

B = 4096
L = 50
D = 128
LANES = 16
NVREG = D // LANES
NBUF = 7


@functools.cache
def _build():
    info = plsc.get_sparse_core_info()
    nw = info.num_cores * info.num_subcores
    b_per_w = B // nw
    n_main = (L - NBUF) // NBUF * NBUF
    mesh = plsc.VectorSubcoreMesh(core_axis_name="c", subcore_axis_name="s")

    @functools.partial(
        pl.kernel,
        mesh=mesh,
        compiler_params=pltpu.CompilerParams(needs_layout_passes=False),
        out_type=jax.ShapeDtypeStruct((B, D), jnp.float32),
        scratch_types=[
            pltpu.VMEM((b_per_w // 2, L), jnp.int32),
            pltpu.VMEM((L * b_per_w,), jnp.int32),
            *[pltpu.VMEM((b_per_w, D), jnp.float32) for _ in range(NBUF)],
            [pltpu.SemaphoreType.DMA] * NBUF,
        ],
    )
    def k(ids_hbm, table_hbm, out_hbm, idx_v, idxt_v, *accs_and_sems):
        accs = accs_and_sems[:NBUF]
        sems = accs_and_sems[NBUF]
        cid = lax.axis_index("c")
        sid = lax.axis_index("s")
        wid = sid * info.num_cores + cid
        base_row = wid * b_per_w

        half = b_per_w // 2

        def transpose_col_half(l, h):
            col = jnp.zeros((LANES,), jnp.int32) + l
            for g in range(half // LANES):
                rows = lax.iota(jnp.int32, LANES) + g * LANES
                vals = plsc.load_gather(idx_v, [rows, col])
                idxt_v[pl.ds(l * b_per_w + h * half + g * LANES, LANES)] = (
                    vals)

        pltpu.sync_copy(ids_hbm.at[pl.ds(base_row, half)], idx_v)

        @pl.loop(0, L)
        def _transpose_h0(l):
            transpose_col_half(l, 0)

        pltpu.sync_copy(ids_hbm.at[pl.ds(base_row + half, half)], idx_v)

        def transpose_col(l):
            transpose_col_half(l, 1)

        def gather(l, buf):
            return pltpu.make_async_copy(
                table_hbm.at[idxt_v.at[pl.ds(l * b_per_w, b_per_w)]],
                accs[buf],
                sems[buf],
            )

        for n in range(NBUF):
            transpose_col(n)
            gather(n, n).start()

        @pl.loop(NBUF, L)
        def _transpose(l):
            transpose_col(l)

        @pl.loop(NBUF, NBUF + n_main, step=NBUF)
        def _gathers(l):
            for n in range(NBUF):
                gather(l + n, n).wait()
                gather(l + n, n).start(add=True)

        for l in range(NBUF + n_main, L):
            n = l % NBUF
            gather(l, n).wait()
            gather(l, n).start(add=True)

        for n in range(NBUF):
            gather(n, n).wait()

        @pl.loop(0, b_per_w)
        def _fold(i):
            for j in range(NVREG):
                s = accs[0][i, pl.ds(j * LANES, LANES)]
                for n in range(1, NBUF):
                    s = s + accs[n][i, pl.ds(j * LANES, LANES)]
                accs[0][i, pl.ds(j * LANES, LANES)] = s * (1.0 / L)

        pltpu.sync_copy(accs[0], out_hbm.at[pl.ds(base_row, b_per_w)])

    return k


def kernel(text_ids, table):
    return _build()(text_ids.astype(jnp.int32), table)


# speedup vs baseline: 1.0374x; 1.0374x over previous
import functools

import jax
import jax.numpy as jnp
from jax import lax
from jax.experimental import pallas as pl
from jax.experimental.pallas import tpu as pltpu
from jax.experimental.pallas import tpu_sc as plsc

B = 4096
L = 50
D = 128
LANES = 16
NVREG = D // LANES
NBUF = 6


@functools.cache
def _build():
    info = plsc.get_sparse_core_info()
    nw = info.num_cores * info.num_subcores
    b_per_w = B // nw
    n_main = (L - NBUF) // NBUF * NBUF
    mesh = plsc.VectorSubcoreMesh(core_axis_name="c", subcore_axis_name="s")

    @functools.partial(
        pl.kernel,
        mesh=mesh,
        compiler_params=pltpu.CompilerParams(needs_layout_passes=False),
        out_type=jax.ShapeDtypeStruct((B, D), jnp.float32),
        scratch_types=[
            pltpu.VMEM((b_per_w, L), jnp.int32),
            pltpu.VMEM((L * b_per_w,), jnp.int32),
            *[pltpu.VMEM((b_per_w, D), jnp.float32) for _ in range(NBUF)],
            [pltpu.SemaphoreType.DMA] * NBUF,
        ],
    )
    def k(ids_hbm, table_hbm, out_hbm, idx_v, idxt_v, *accs_and_sems):
        accs = accs_and_sems[:NBUF]
        sems = accs_and_sems[NBUF]
        cid = lax.axis_index("c")
        sid = lax.axis_index("s")
        wid = sid * info.num_cores + cid
        base_row = wid * b_per_w

        pltpu.sync_copy(ids_hbm.at[pl.ds(base_row, b_per_w)], idx_v)

        def transpose_col(l):
            col = jnp.zeros((LANES,), jnp.int32) + l
            for g in range(b_per_w // LANES):
                rows = lax.iota(jnp.int32, LANES) + g * LANES
                vals = plsc.load_gather(idx_v, [rows, col])
                idxt_v[pl.ds(l * b_per_w + g * LANES, LANES)] = vals

        def gather(l, buf):
            return pltpu.make_async_copy(
                table_hbm.at[idxt_v.at[pl.ds(l * b_per_w, b_per_w)]],
                accs[buf],
                sems[buf],
            )

        for n in range(NBUF):
            transpose_col(n)
            gather(n, n).start()

        @pl.loop(NBUF, L)
        def _transpose(l):
            transpose_col(l)

        @pl.loop(NBUF, NBUF + n_main, step=NBUF)
        def _gathers(l):
            for n in range(NBUF):
                gather(l + n, n).wait()
                gather(l + n, n).start(add=True)

        for l in range(NBUF + n_main, L):
            n = l % NBUF
            gather(l, n).wait()
            gather(l, n).start(add=True)

        for n in range(NBUF):
            gather(n, n).wait()

        @pl.loop(0, b_per_w)
        def _fold(i):
            for j in range(NVREG):
                s = accs[0][i, pl.ds(j * LANES, LANES)]
                for n in range(1, NBUF):
                    s = s + accs[n][i, pl.ds(j * LANES, LANES)]
                accs[0][i, pl.ds(j * LANES, LANES)] = s * (1.0 / L)

        pltpu.sync_copy(accs[0], out_hbm.at[pl.ds(base_row, b_per_w)])

    return k


def kernel(text_ids, table):
    return _build()(text_ids.astype(jnp.int32), table)
